# baseline (speedup 1.0000x reference)
import functools

import jax
import jax.numpy as jnp
from jax import lax
from jax.experimental import pallas as pl
from jax.experimental.pallas import tpu as pltpu
from jax.experimental.pallas import tpu_sc as plsc

_VOCAB = 100000
_BATCH = 16384
_SEQ = 200

_INFO = plsc.get_sparse_core_info()
_NC, _NS, _L = _INFO.num_cores, _INFO.num_subcores, _INFO.num_lanes
_NW = _NC * _NS
_COLS_W = _BATCH // _NW
_ROWS_C = 8
_NCHUNK = _SEQ // _ROWS_C


def _vocab_body(
    tok_hbm, table_hbm, out_hbm,
    table_v, idx_v0, idx_v1, out_v0, out_v1,
    sem_t, sem_i0, sem_i1, sem_o0, sem_o1,
):
    idxs, outs = (idx_v0, idx_v1), (out_v0, out_v1)
    sem_i, sem_o = (sem_i0, sem_i1), (sem_o0, sem_o1)
    wid = lax.axis_index("s") * _NC + lax.axis_index("c")
    col0 = wid * _COLS_W

    def tok_slice(c):
        return tok_hbm.at[pl.ds(c * _ROWS_C, _ROWS_C), pl.ds(col0, _COLS_W)]

    def out_slice(c):
        return out_hbm.at[pl.ds(c * _ROWS_C, _ROWS_C), pl.ds(col0, _COLS_W)]

    def gather_chunk(b):
        @plsc.parallel_loop(0, _ROWS_C, step=1, unroll=8)
        def _rows(r):
            for off in range(0, _COLS_W, _L):
                idx = idxs[b][r, pl.ds(off, _L)]
                outs[b][r, pl.ds(off, _L)] = plsc.load_gather(table_v, [idx])

    pltpu.async_copy(tok_slice(0), idxs[0], sem_i[0])
    pltpu.async_copy(tok_slice(1), idxs[1], sem_i[1])
    pltpu.async_copy(table_hbm, table_v, sem_t).wait()

    def pair_body(p, carry):
        for b in (0, 1):
            c = 2 * p + b
            pltpu.make_async_copy(tok_slice(c), idxs[b], sem_i[b]).wait()

            @pl.when(p >= 1)
            def _wait_out():
                pltpu.make_async_copy(outs[b], out_slice(c - 2), sem_o[b]).wait()

            gather_chunk(b)
            pltpu.async_copy(outs[b], out_slice(c), sem_o[b])

            if b == 0:
                pltpu.async_copy(tok_slice(c + 2), idxs[b], sem_i[b])
            else:
                @pl.when(p <= (_NCHUNK - 4) // 2)
                def _next_in():
                    pltpu.async_copy(tok_slice(c + 2), idxs[b], sem_i[b])

        return carry

    lax.fori_loop(0, (_NCHUNK - 1) // 2, pair_body, 0)

    c_last = _NCHUNK - 1
    pltpu.make_async_copy(tok_slice(c_last), idxs[0], sem_i[0]).wait()
    pltpu.make_async_copy(outs[0], out_slice(c_last - 2), sem_o[0]).wait()
    gather_chunk(0)
    pltpu.async_copy(outs[0], out_slice(c_last), sem_o[0])

    pltpu.make_async_copy(outs[1], out_slice(c_last - 1), sem_o[1]).wait()
    pltpu.make_async_copy(outs[0], out_slice(c_last), sem_o[0]).wait()


@jax.jit
def _lookup(tok_t, vocab_table):
    mesh = plsc.VectorSubcoreMesh(core_axis_name="c", subcore_axis_name="s")
    run = pl.kernel(
        _vocab_body,
        mesh=mesh,
        out_type=jax.ShapeDtypeStruct((_SEQ, _BATCH), jnp.float32),
        scratch_types=[
            pltpu.VMEM((_VOCAB,), jnp.float32),
            pltpu.VMEM((_ROWS_C, _COLS_W), jnp.int32),
            pltpu.VMEM((_ROWS_C, _COLS_W), jnp.int32),
            pltpu.VMEM((_ROWS_C, _COLS_W), jnp.float32),
            pltpu.VMEM((_ROWS_C, _COLS_W), jnp.float32),
            pltpu.SemaphoreType.DMA,
            pltpu.SemaphoreType.DMA,
            pltpu.SemaphoreType.DMA,
            pltpu.SemaphoreType.DMA,
            pltpu.SemaphoreType.DMA,
        ],
        compiler_params=pltpu.CompilerParams(needs_layout_passes=False),
    )
    return run(tok_t, vocab_table)


def kernel(tokens_list, vocab_table):
    return _lookup(tokens_list.T, vocab_table).T


# speedup vs baseline: 1.0697x; 1.0697x over previous
import functools

import jax
import jax.numpy as jnp
from jax import lax
from jax.experimental import pallas as pl
from jax.experimental.pallas import tpu as pltpu
from jax.experimental.pallas import tpu_sc as plsc

_VOCAB = 100000
_BATCH = 16384
_SEQ = 200

_INFO = plsc.get_sparse_core_info()
_NC, _NS, _L = _INFO.num_cores, _INFO.num_subcores, _INFO.num_lanes
_NW = _NC * _NS
_COLS_W = _BATCH // _NW
_ROWS_C = 8
_NCHUNK = _SEQ // _ROWS_C


def _vocab_body(
    tok_hbm, table_hbm, out_hbm,
    table_v, idx_v0, idx_v1, out_v0, out_v1,
    sem_t, sem_i0, sem_i1, sem_o0, sem_o1,
):
    idxs, outs = (idx_v0, idx_v1), (out_v0, out_v1)
    sem_i, sem_o = (sem_i0, sem_i1), (sem_o0, sem_o1)
    wid = lax.axis_index("s") * _NC + lax.axis_index("c")
    col0 = wid * _COLS_W

    def tok_slice(c):
        return tok_hbm.at[pl.ds(c * _ROWS_C, _ROWS_C), pl.ds(col0, _COLS_W)]

    def out_slice(c):
        return out_hbm.at[pl.ds(c * _ROWS_C, _ROWS_C), pl.ds(col0, _COLS_W)]

    def gather_chunk(b):
        @plsc.parallel_loop(0, _ROWS_C, step=1, unroll=4)
        def _rows(r):
            for off in range(0, _COLS_W, _L):
                idx = idxs[b][r, pl.ds(off, _L)]
                outs[b][r, pl.ds(off, _L)] = plsc.load_gather(table_v, [idx])

    pltpu.async_copy(tok_slice(0), idxs[0], sem_i[0])
    pltpu.async_copy(tok_slice(1), idxs[1], sem_i[1])
    pltpu.async_copy(table_hbm, table_v, sem_t).wait()

    def pair_body(p, carry):
        for b in (0, 1):
            c = 2 * p + b
            pltpu.make_async_copy(tok_slice(c), idxs[b], sem_i[b]).wait()

            @pl.when(p >= 1)
            def _wait_out():
                pltpu.make_async_copy(outs[b], out_slice(c - 2), sem_o[b]).wait()

            gather_chunk(b)
            pltpu.async_copy(outs[b], out_slice(c), sem_o[b])

            if b == 0:
                pltpu.async_copy(tok_slice(c + 2), idxs[b], sem_i[b])
            else:
                @pl.when(p <= (_NCHUNK - 4) // 2)
                def _next_in():
                    pltpu.async_copy(tok_slice(c + 2), idxs[b], sem_i[b])

        return carry

    lax.fori_loop(0, (_NCHUNK - 1) // 2, pair_body, 0)

    c_last = _NCHUNK - 1
    pltpu.make_async_copy(tok_slice(c_last), idxs[0], sem_i[0]).wait()
    pltpu.make_async_copy(outs[0], out_slice(c_last - 2), sem_o[0]).wait()
    gather_chunk(0)
    pltpu.async_copy(outs[0], out_slice(c_last), sem_o[0])

    pltpu.make_async_copy(outs[1], out_slice(c_last - 1), sem_o[1]).wait()
    pltpu.make_async_copy(outs[0], out_slice(c_last), sem_o[0]).wait()


@jax.jit
def _lookup(tok_t, vocab_table):
    mesh = plsc.VectorSubcoreMesh(core_axis_name="c", subcore_axis_name="s")
    run = pl.kernel(
        _vocab_body,
        mesh=mesh,
        out_type=jax.ShapeDtypeStruct((_SEQ, _BATCH), jnp.float32),
        scratch_types=[
            pltpu.VMEM((_VOCAB,), jnp.float32),
            pltpu.VMEM((_ROWS_C, _COLS_W), jnp.int32),
            pltpu.VMEM((_ROWS_C, _COLS_W), jnp.int32),
            pltpu.VMEM((_ROWS_C, _COLS_W), jnp.float32),
            pltpu.VMEM((_ROWS_C, _COLS_W), jnp.float32),
            pltpu.SemaphoreType.DMA,
            pltpu.SemaphoreType.DMA,
            pltpu.SemaphoreType.DMA,
            pltpu.SemaphoreType.DMA,
            pltpu.SemaphoreType.DMA,
        ],
        compiler_params=pltpu.CompilerParams(needs_layout_passes=False),
    )
    return run(tok_t, vocab_table)


def kernel(tokens_list, vocab_table):
    return _lookup(tokens_list.T, vocab_table).T
